# baseline (speedup 1.0000x reference)
import functools

import jax
import jax.numpy as jnp
from jax import lax
from jax.experimental import pallas as pl
from jax.experimental.pallas import tpu as pltpu
from jax.experimental.pallas import tpu_sc as plsc

_EMBED_DIM = 64
_GROUP = 128
_NBUF = 5


def _mesh_info():
  info = plsc.get_sparse_core_info()
  return info.num_cores, info.num_subcores


_PANEL = 256


def _make_relayout(vocab: int):
  nc, ns = _mesh_info()
  nw = nc * ns
  full_cols = vocab // _PANEL
  rem = vocab - full_cols * _PANEL
  base_cnt = (full_cols // nw) & ~1
  n_extra = full_cols - base_cnt * nw

  mesh = plsc.VectorSubcoreMesh(core_axis_name="c", subcore_axis_name="s")

  @functools.partial(
      pl.kernel,
      mesh=mesh,
      out_type=jax.ShapeDtypeStruct((vocab * _EMBED_DIM,), jnp.float32),
      scratch_types=[
          pltpu.VMEM((_EMBED_DIM, _PANEL), jnp.float32),
          pltpu.VMEM((_EMBED_DIM, _PANEL), jnp.float32),
          pltpu.VMEM((_PANEL * _EMBED_DIM,), jnp.float32),
          pltpu.VMEM((_PANEL * _EMBED_DIM,), jnp.float32),
          pltpu.VMEM((_EMBED_DIM, _EMBED_DIM), jnp.float32),
          pltpu.SemaphoreType.DMA,
          pltpu.SemaphoreType.DMA,
      ],
      compiler_params=pltpu.CompilerParams(needs_layout_passes=False),
  )
  def relayout(tab_t, out_hbm, vbuf0, vbuf1, obuf0, obuf1, tbuf, isem, osem):
    wid = lax.axis_index("s") * nc + lax.axis_index("c")
    vbufs = (vbuf0, vbuf1)
    obufs = (obuf0, obuf1)

    def col(i):
      return wid + nw * i

    def fire_in(ct, b):
      pltpu.async_copy(tab_t.at[:, pl.ds(ct * _PANEL, _PANEL)],
                       vbufs[b], isem)

    def wait_in(ct, b):
      pltpu.make_async_copy(tab_t.at[:, pl.ds(ct * _PANEL, _PANEL)],
                            vbufs[b], isem).wait()

    def fire_out(ct, b):
      pltpu.async_copy(obufs[b],
                       out_hbm.at[pl.ds(ct * (_PANEL * _EMBED_DIM),
                                        _PANEL * _EMBED_DIM)], osem)

    def wait_out(ct, b):
      pltpu.make_async_copy(obufs[b],
                            out_hbm.at[pl.ds(ct * (_PANEL * _EMBED_DIM),
                                             _PANEL * _EMBED_DIM)],
                            osem).wait()

    def transpose(b, width=_PANEL):
      lanes = lax.iota(jnp.int32, 16)
      vbuf, obuf = vbufs[b], obufs[b]

      @plsc.parallel_loop(0, width // 16, unroll=8)
      def vchunk(k):
        v_vec = lanes + 16 * k
        out0 = v_vec * _EMBED_DIM
        for j in range(_EMBED_DIM // 16):
          for s in range(16):
            e_vec = 16 * j + ((lanes + s) & 15)
            vals = plsc.load_gather(vbuf, [e_vec, v_vec])
            plsc.store_scatter(obuf, [out0 + e_vec], vals)

    fire_in(col(0), 0)

    def pair(k, carry):
      i0 = 2 * k
      fire_in(col(i0 + 1), 1)
      wait_in(col(i0), 0)
      pl.when(k >= 1)(lambda: wait_out(col(i0 - 2), 0))
      transpose(0)
      fire_out(col(i0), 0)
      pl.when(k < base_cnt // 2 - 1)(lambda: fire_in(col(i0 + 2), 0))
      wait_in(col(i0 + 1), 1)
      pl.when(k >= 1)(lambda: wait_out(col(i0 - 1), 1))
      transpose(1)
      fire_out(col(i0 + 1), 1)
      return carry

    lax.fori_loop(0, base_cnt // 2, pair, 0)
    wait_out(col(base_cnt - 2), 0)
    wait_out(col(base_cnt - 1), 1)

    if n_extra:
      @pl.when(wid < n_extra)
      def _():
        ct = base_cnt * nw + wid
        pltpu.sync_copy(tab_t.at[:, pl.ds(ct * _PANEL, _PANEL)], vbuf0)
        transpose(0)
        pltpu.sync_copy(obuf0,
                        out_hbm.at[pl.ds(ct * (_PANEL * _EMBED_DIM),
                                         _PANEL * _EMBED_DIM)])

    if rem:
      @pl.when(wid == nw - 1)
      def _():
        lanes = lax.iota(jnp.int32, 16)
        pltpu.sync_copy(tab_t.at[:, pl.ds(full_cols * _PANEL, rem)], tbuf)

        @plsc.parallel_loop(0, rem // 16, unroll=2)
        def vchunk(k):
          v_vec = lanes + 16 * k
          out0 = v_vec * _EMBED_DIM
          for j in range(_EMBED_DIM // 16):
            for s in range(16):
              e_vec = 16 * j + ((lanes + s) & 15)
              vals = plsc.load_gather(tbuf, [e_vec, v_vec])
              plsc.store_scatter(obuf1, [out0 + e_vec], vals)
        pltpu.sync_copy(obuf1.at[pl.ds(0, rem * _EMBED_DIM)],
                        out_hbm.at[pl.ds(full_cols * _PANEL * _EMBED_DIM,
                                         rem * _EMBED_DIM)])

  return relayout


def _make_gather(num_groups: int):
  nc, ns = _mesh_info()
  nw = nc * ns
  assert num_groups % nw == 0
  gpw = num_groups // nw
  assert gpw % _NBUF == 0

  mesh = plsc.VectorSubcoreMesh(core_axis_name="c", subcore_axis_name="s")

  @functools.partial(
      pl.kernel,
      mesh=mesh,
      out_type=jax.ShapeDtypeStruct((num_groups * _GROUP, _EMBED_DIM),
                                    jnp.float32),
      scratch_types=[
          pltpu.VMEM((gpw, _GROUP), jnp.int32),
          pltpu.VMEM((_NBUF, _GROUP, _EMBED_DIM), jnp.float32),
          pltpu.SemaphoreType.DMA,
          pltpu.SemaphoreType.DMA,
      ],
      compiler_params=pltpu.CompilerParams(use_tc_tiling_on_sc=False),
  )
  def gather_kernel(idx_hbm, table_hbm, out_hbm, idx_v, rows_v, gsem, wsem):
    wid = lax.axis_index("s") * nc + lax.axis_index("c")
    g0 = wid * gpw
    pltpu.sync_copy(idx_hbm.at[wid], idx_v)

    def out_base(j):
      g = g0 + j
      return (8 * (g // 64) + g % 8) * 1024 + 128 * ((g % 64) // 8)

    def fire_g(j, b):
      pltpu.async_copy(table_hbm.at[idx_v.at[j]], rows_v.at[b], gsem)

    def wait_g(j, b):
      pltpu.make_async_copy(table_hbm.at[idx_v.at[j]],
                            rows_v.at[b], gsem).wait()

    def fire_wb(j, b):
      pltpu.async_copy(rows_v.at[b],
                       out_hbm.at[pl.ds(out_base(j), _GROUP)], wsem)

    def wait_wb(j, b):
      pltpu.make_async_copy(rows_v.at[b],
                            out_hbm.at[pl.ds(out_base(j), _GROUP)],
                            wsem).wait()

    for b in range(_NBUF):
      fire_g(b, b)

    def step(k, carry):
      for b in range(_NBUF):
        j = _NBUF * k + b
        wait_g(j, b)
        fire_wb(j, b)
        wait_wb(j, b)
        fire_g(j + _NBUF, b)
      return carry

    lax.fori_loop(0, gpw // _NBUF - 1, step, 0)

    for b in range(_NBUF):
      j = gpw - _NBUF + b
      wait_g(j, b)
      fire_wb(j, b)
      wait_wb(j, b)

  return gather_kernel


def kernel(input_ids, word_embeddings):
  batch, seq = input_ids.shape
  vocab, dim = word_embeddings.shape
  n = batch * seq
  num_groups = n // _GROUP
  nw = 32
  idx = (input_ids.T.reshape(seq // 8, 8, batch // _GROUP, _GROUP)
         .transpose(0, 2, 1, 3)
         .reshape(nw, num_groups // nw, _GROUP))
  table_lin = _make_relayout(vocab)(word_embeddings.T)
  table = table_lin.reshape(vocab, dim)
  out = _make_gather(num_groups)(idx, table)
  return out.reshape(seq, batch, dim).transpose(1, 0, 2)


# speedup vs baseline: 1.5788x; 1.5788x over previous
import functools

import jax
import jax.numpy as jnp
from jax import lax
from jax.experimental import pallas as pl
from jax.experimental.pallas import tpu as pltpu
from jax.experimental.pallas import tpu_sc as plsc

_EMBED_DIM = 64
_GROUP = 128
_NBUF = 5


def _mesh_info():
  info = plsc.get_sparse_core_info()
  return info.num_cores, info.num_subcores


_PANEL = 256


def _make_relayout(vocab: int):
  nc, ns = _mesh_info()
  nw = nc * ns
  full_cols = vocab // _PANEL
  rem = vocab - full_cols * _PANEL
  base_cnt = (full_cols // nw) & ~1
  n_extra = full_cols - base_cnt * nw

  mesh = plsc.VectorSubcoreMesh(core_axis_name="c", subcore_axis_name="s")

  @functools.partial(
      pl.kernel,
      mesh=mesh,
      out_type=jax.ShapeDtypeStruct((vocab * _EMBED_DIM,), jnp.float32),
      scratch_types=[
          pltpu.VMEM((_EMBED_DIM, _PANEL), jnp.float32),
          pltpu.VMEM((_EMBED_DIM, _PANEL), jnp.float32),
          pltpu.VMEM((_PANEL * _EMBED_DIM,), jnp.float32),
          pltpu.VMEM((_PANEL * _EMBED_DIM,), jnp.float32),
          pltpu.VMEM((_EMBED_DIM, _EMBED_DIM), jnp.float32),
          pltpu.SemaphoreType.DMA,
          pltpu.SemaphoreType.DMA,
      ],
      compiler_params=pltpu.CompilerParams(needs_layout_passes=False),
  )
  def relayout(tab_t, out_hbm, vbuf0, vbuf1, obuf0, obuf1, tbuf, isem, osem):
    wid = lax.axis_index("s") * nc + lax.axis_index("c")
    vbufs = (vbuf0, vbuf1)
    obufs = (obuf0, obuf1)

    def col(i):
      return wid + nw * i

    def fire_in(ct, b):
      pltpu.async_copy(tab_t.at[:, pl.ds(ct * _PANEL, _PANEL)],
                       vbufs[b], isem)

    def wait_in(ct, b):
      pltpu.make_async_copy(tab_t.at[:, pl.ds(ct * _PANEL, _PANEL)],
                            vbufs[b], isem).wait()

    def fire_out(ct, b):
      pltpu.async_copy(obufs[b],
                       out_hbm.at[pl.ds(ct * (_PANEL * _EMBED_DIM),
                                        _PANEL * _EMBED_DIM)], osem)

    def wait_out(ct, b):
      pltpu.make_async_copy(obufs[b],
                            out_hbm.at[pl.ds(ct * (_PANEL * _EMBED_DIM),
                                             _PANEL * _EMBED_DIM)],
                            osem).wait()

    def transpose(b, width=_PANEL):
      lanes = lax.iota(jnp.int32, 16)
      vbuf, obuf = vbufs[b], obufs[b]

      @plsc.parallel_loop(0, width // 16, unroll=4)
      def vchunk(k):
        v_vec = lanes + 16 * k
        out0 = v_vec * _EMBED_DIM
        for j in range(_EMBED_DIM // 16):
          for s in range(16):
            e_vec = 16 * j + ((lanes + s) & 15)
            vals = plsc.load_gather(vbuf, [e_vec, v_vec])
            plsc.store_scatter(obuf, [out0 + e_vec], vals)

    fire_in(col(0), 0)

    def pair(k, carry):
      i0 = 2 * k
      fire_in(col(i0 + 1), 1)
      wait_in(col(i0), 0)
      pl.when(k >= 1)(lambda: wait_out(col(i0 - 2), 0))
      transpose(0)
      fire_out(col(i0), 0)
      pl.when(k < base_cnt // 2 - 1)(lambda: fire_in(col(i0 + 2), 0))
      wait_in(col(i0 + 1), 1)
      pl.when(k >= 1)(lambda: wait_out(col(i0 - 1), 1))
      transpose(1)
      fire_out(col(i0 + 1), 1)
      return carry

    lax.fori_loop(0, base_cnt // 2, pair, 0)
    wait_out(col(base_cnt - 2), 0)
    wait_out(col(base_cnt - 1), 1)

    if n_extra:
      @pl.when(wid < n_extra)
      def _():
        ct = base_cnt * nw + wid
        pltpu.sync_copy(tab_t.at[:, pl.ds(ct * _PANEL, _PANEL)], vbuf0)
        transpose(0)
        pltpu.sync_copy(obuf0,
                        out_hbm.at[pl.ds(ct * (_PANEL * _EMBED_DIM),
                                         _PANEL * _EMBED_DIM)])

    if rem:
      @pl.when(wid == nw - 1)
      def _():
        lanes = lax.iota(jnp.int32, 16)
        pltpu.sync_copy(tab_t.at[:, pl.ds(full_cols * _PANEL, rem)], tbuf)

        @plsc.parallel_loop(0, rem // 16, unroll=2)
        def vchunk(k):
          v_vec = lanes + 16 * k
          out0 = v_vec * _EMBED_DIM
          for j in range(_EMBED_DIM // 16):
            for s in range(16):
              e_vec = 16 * j + ((lanes + s) & 15)
              vals = plsc.load_gather(tbuf, [e_vec, v_vec])
              plsc.store_scatter(obuf1, [out0 + e_vec], vals)
        pltpu.sync_copy(obuf1.at[pl.ds(0, rem * _EMBED_DIM)],
                        out_hbm.at[pl.ds(full_cols * _PANEL * _EMBED_DIM,
                                         rem * _EMBED_DIM)])

  return relayout


def _make_gather(num_groups: int, seq: int, batch: int):
  nc, ns = _mesh_info()
  nw = nc * ns
  assert num_groups % nw == 0
  gpw = num_groups // nw
  assert gpw % _NBUF == 0

  mesh = plsc.VectorSubcoreMesh(core_axis_name="c", subcore_axis_name="s")

  @functools.partial(
      pl.kernel,
      mesh=mesh,
      out_type=jax.ShapeDtypeStruct(
          (seq, _EMBED_DIM // 8, batch // _GROUP, 8, _GROUP), jnp.float32),
      scratch_types=[
          pltpu.VMEM((gpw, _GROUP), jnp.int32),
      ] + [pltpu.VMEM((_GROUP, _EMBED_DIM), jnp.float32)] * _NBUF
        + [pltpu.VMEM((_EMBED_DIM // 8, 8, _GROUP), jnp.float32)] * _NBUF
        + [
          pltpu.SemaphoreType.DMA,
          pltpu.SemaphoreType.DMA,
      ],
      compiler_params=pltpu.CompilerParams(use_tc_tiling_on_sc=False,
                                          needs_layout_passes=False),
  )
  def gather_kernel(idx_hbm, table_hbm, out_hbm, idx_v, *rest):
    rows = rest[:_NBUF]
    tobs = rest[_NBUF:2 * _NBUF]
    gsem, wsem = rest[2 * _NBUF], rest[2 * _NBUF + 1]
    wid = lax.axis_index("s") * nc + lax.axis_index("c")
    g0 = wid * gpw
    pltpu.sync_copy(idx_hbm.at[wid], idx_v)
    lanes = lax.iota(jnp.int32, 16)

    def out_dst(j):
      g = g0 + j
      sq = 8 * (g // 64) + g % 8
      bt = (g % 64) // 8
      return out_hbm.at[sq, :, bt]

    def fire_g(j, b):
      pltpu.async_copy(table_hbm.at[idx_v.at[j]], rows[b], gsem)

    def wait_g(j, b):
      pltpu.make_async_copy(table_hbm.at[idx_v.at[j]], rows[b], gsem).wait()

    def transpose(b):
      @plsc.parallel_loop(0, _GROUP // 16, unroll=4)
      def vchunk(k):
        v_vec = lanes + 16 * k
        for j in range(_EMBED_DIM // 16):
          for t in range(16):
            e_vec = 16 * j + ((lanes + t) & 15)
            vals = plsc.load_gather(rows[b], [v_vec, e_vec])
            plsc.store_scatter(tobs[b],
                               [e_vec >> 3, e_vec & 7, v_vec], vals)

    def fire_wb(j, b):
      pltpu.async_copy(tobs[b], out_dst(j), wsem)

    def wait_wb(j, b):
      pltpu.make_async_copy(tobs[b], out_dst(j), wsem).wait()

    for b in range(_NBUF):
      fire_g(b, b)

    def step(k, carry):
      for b in range(_NBUF):
        j = _NBUF * k + b
        wait_g(j, b)
        transpose(b)
        fire_wb(j, b)
        wait_wb(j, b)
        fire_g(j + _NBUF, b)
      return carry

    lax.fori_loop(0, gpw // _NBUF - 1, step, 0)

    for b in range(_NBUF):
      j = gpw - _NBUF + b
      wait_g(j, b)
      transpose(b)
      fire_wb(j, b)
      wait_wb(j, b)

  return gather_kernel


def kernel(input_ids, word_embeddings):
  batch, seq = input_ids.shape
  vocab, dim = word_embeddings.shape
  n = batch * seq
  num_groups = n // _GROUP
  nw = 32
  idx = (input_ids.T.reshape(seq // 8, 8, batch // _GROUP, _GROUP)
         .transpose(0, 2, 1, 3)
         .reshape(nw, num_groups // nw, _GROUP))
  table_lin = _make_relayout(vocab)(word_embeddings.T)
  table = table_lin.reshape(vocab, dim)
  out5 = _make_gather(num_groups, seq, batch)(idx, table)
  return (out5.transpose(2, 4, 0, 1, 3)
          .reshape(batch, seq, dim))


# speedup vs baseline: 1.6097x; 1.0196x over previous
import functools

import jax
import jax.numpy as jnp
from jax import lax
from jax.experimental import pallas as pl
from jax.experimental.pallas import tpu as pltpu
from jax.experimental.pallas import tpu_sc as plsc

_EMBED_DIM = 64
_GROUP = 128
_NBUF = 5


def _mesh_info():
  info = plsc.get_sparse_core_info()
  return info.num_cores, info.num_subcores


_PANEL = 256


def _make_relayout(vocab: int):
  nc, ns = _mesh_info()
  nw = nc * ns
  full_cols = vocab // _PANEL
  rem = vocab - full_cols * _PANEL
  base_cnt = (full_cols // nw) & ~1
  n_extra = full_cols - base_cnt * nw

  mesh = plsc.VectorSubcoreMesh(core_axis_name="c", subcore_axis_name="s")

  @functools.partial(
      pl.kernel,
      mesh=mesh,
      out_type=jax.ShapeDtypeStruct((vocab * _EMBED_DIM,), jnp.float32),
      scratch_types=[
          pltpu.VMEM((_EMBED_DIM, _PANEL), jnp.float32),
          pltpu.VMEM((_EMBED_DIM, _PANEL), jnp.float32),
          pltpu.VMEM((_PANEL * _EMBED_DIM,), jnp.float32),
          pltpu.VMEM((_PANEL * _EMBED_DIM,), jnp.float32),
          pltpu.VMEM((_EMBED_DIM, _EMBED_DIM), jnp.float32),
          pltpu.SemaphoreType.DMA,
          pltpu.SemaphoreType.DMA,
      ],
      compiler_params=pltpu.CompilerParams(needs_layout_passes=False),
  )
  def relayout(tab_t, out_hbm, vbuf0, vbuf1, obuf0, obuf1, tbuf, isem, osem):
    wid = lax.axis_index("s") * nc + lax.axis_index("c")
    vbufs = (vbuf0, vbuf1)
    obufs = (obuf0, obuf1)

    def col(i):
      return wid + nw * i

    def fire_in(ct, b):
      pltpu.async_copy(tab_t.at[:, pl.ds(ct * _PANEL, _PANEL)],
                       vbufs[b], isem)

    def wait_in(ct, b):
      pltpu.make_async_copy(tab_t.at[:, pl.ds(ct * _PANEL, _PANEL)],
                            vbufs[b], isem).wait()

    def fire_out(ct, b):
      pltpu.async_copy(obufs[b],
                       out_hbm.at[pl.ds(ct * (_PANEL * _EMBED_DIM),
                                        _PANEL * _EMBED_DIM)], osem)

    def wait_out(ct, b):
      pltpu.make_async_copy(obufs[b],
                            out_hbm.at[pl.ds(ct * (_PANEL * _EMBED_DIM),
                                             _PANEL * _EMBED_DIM)],
                            osem).wait()

    def transpose(b, width=_PANEL):
      lanes = lax.iota(jnp.int32, 16)
      vbuf, obuf = vbufs[b], obufs[b]

      @plsc.parallel_loop(0, width // 16, unroll=4)
      def vchunk(k):
        v_vec = lanes + 16 * k
        out0 = v_vec * _EMBED_DIM
        for j in range(_EMBED_DIM // 16):
          for s in range(16):
            e_vec = 16 * j + ((lanes + s) & 15)
            vals = plsc.load_gather(vbuf, [e_vec, v_vec])
            plsc.store_scatter(obuf, [out0 + e_vec], vals)

    fire_in(col(0), 0)

    def pair(k, carry):
      i0 = 2 * k
      fire_in(col(i0 + 1), 1)
      wait_in(col(i0), 0)
      pl.when(k >= 1)(lambda: wait_out(col(i0 - 2), 0))
      transpose(0)
      fire_out(col(i0), 0)
      pl.when(k < base_cnt // 2 - 1)(lambda: fire_in(col(i0 + 2), 0))
      wait_in(col(i0 + 1), 1)
      pl.when(k >= 1)(lambda: wait_out(col(i0 - 1), 1))
      transpose(1)
      fire_out(col(i0 + 1), 1)
      return carry

    lax.fori_loop(0, base_cnt // 2, pair, 0)
    wait_out(col(base_cnt - 2), 0)
    wait_out(col(base_cnt - 1), 1)

    if n_extra:
      @pl.when(wid < n_extra)
      def _():
        ct = base_cnt * nw + wid
        pltpu.sync_copy(tab_t.at[:, pl.ds(ct * _PANEL, _PANEL)], vbuf0)
        transpose(0)
        pltpu.sync_copy(obuf0,
                        out_hbm.at[pl.ds(ct * (_PANEL * _EMBED_DIM),
                                         _PANEL * _EMBED_DIM)])

    if rem:
      @pl.when(wid == nw - 1)
      def _():
        lanes = lax.iota(jnp.int32, 16)
        pltpu.sync_copy(tab_t.at[:, pl.ds(full_cols * _PANEL, rem)], tbuf)

        @plsc.parallel_loop(0, rem // 16, unroll=2)
        def vchunk(k):
          v_vec = lanes + 16 * k
          out0 = v_vec * _EMBED_DIM
          for j in range(_EMBED_DIM // 16):
            for s in range(16):
              e_vec = 16 * j + ((lanes + s) & 15)
              vals = plsc.load_gather(tbuf, [e_vec, v_vec])
              plsc.store_scatter(obuf1, [out0 + e_vec], vals)
        pltpu.sync_copy(obuf1.at[pl.ds(0, rem * _EMBED_DIM)],
                        out_hbm.at[pl.ds(full_cols * _PANEL * _EMBED_DIM,
                                         rem * _EMBED_DIM)])

  return relayout


def _make_gather(num_groups: int, seq: int, batch: int):
  nc, ns = _mesh_info()
  nw = nc * ns
  assert num_groups % nw == 0
  gpw = num_groups // nw
  assert gpw % _NBUF == 0

  mesh = plsc.VectorSubcoreMesh(core_axis_name="c", subcore_axis_name="s")

  @functools.partial(
      pl.kernel,
      mesh=mesh,
      out_type=jax.ShapeDtypeStruct(
          (seq, _EMBED_DIM // 8, batch // _GROUP, 8, _GROUP), jnp.float32),
      scratch_types=[
          pltpu.VMEM((gpw, _GROUP), jnp.int32),
      ] + [pltpu.VMEM((_GROUP, _EMBED_DIM), jnp.float32)] * _NBUF
        + [pltpu.VMEM((_EMBED_DIM // 8, 8, _GROUP), jnp.float32)] * _NBUF
        + [
          pltpu.SemaphoreType.DMA,
          pltpu.SemaphoreType.DMA,
      ],
      compiler_params=pltpu.CompilerParams(use_tc_tiling_on_sc=False,
                                          needs_layout_passes=False),
  )
  def gather_kernel(idx_hbm, table_hbm, out_hbm, idx_v, *rest):
    rows = rest[:_NBUF]
    tobs = rest[_NBUF:2 * _NBUF]
    gsem, wsem = rest[2 * _NBUF], rest[2 * _NBUF + 1]
    wid = lax.axis_index("s") * nc + lax.axis_index("c")
    g0 = wid * gpw
    pltpu.sync_copy(idx_hbm.at[wid], idx_v)
    lanes = lax.iota(jnp.int32, 16)

    def out_dst(j):
      g = g0 + j
      sq = 8 * (g // 64) + g % 8
      bt = (g % 64) // 8
      return out_hbm.at[sq, :, bt]

    def fire_g(j, b):
      pltpu.async_copy(table_hbm.at[idx_v.at[j]], rows[b], gsem)

    def wait_g(j, b):
      pltpu.make_async_copy(table_hbm.at[idx_v.at[j]], rows[b], gsem).wait()

    def transpose(b):
      @plsc.parallel_loop(0, _GROUP // 16, unroll=4)
      def vchunk(k):
        v_vec = lanes + 16 * k
        for j in range(_EMBED_DIM // 16):
          for t in range(16):
            e_vec = 16 * j + ((lanes + t) & 15)
            vals = plsc.load_gather(rows[b], [v_vec, e_vec])
            plsc.store_scatter(tobs[b],
                               [e_vec >> 3, e_vec & 7, v_vec], vals)

    def fire_wb(j, b):
      pltpu.async_copy(tobs[b], out_dst(j), wsem)

    def wait_wb(j, b):
      pltpu.make_async_copy(tobs[b], out_dst(j), wsem).wait()

    for b in range(_NBUF):
      fire_g(b, b)

    def step(k, carry):
      for b in range(_NBUF):
        j = _NBUF * k + b
        wait_g(j, b)
        pl.when(k >= 1)(lambda: wait_wb(j - _NBUF, b))
        transpose(b)
        fire_wb(j, b)
        fire_g(j + _NBUF, b)
      return carry

    lax.fori_loop(0, gpw // _NBUF - 1, step, 0)

    for b in range(_NBUF):
      j = gpw - _NBUF + b
      wait_g(j, b)
      wait_wb(j - _NBUF, b)
      transpose(b)
      fire_wb(j, b)
    for b in range(_NBUF):
      wait_wb(gpw - _NBUF + b, b)

  return gather_kernel


def kernel(input_ids, word_embeddings):
  batch, seq = input_ids.shape
  vocab, dim = word_embeddings.shape
  n = batch * seq
  num_groups = n // _GROUP
  nw = 32
  idx = (input_ids.T.reshape(seq // 8, 8, batch // _GROUP, _GROUP)
         .transpose(0, 2, 1, 3)
         .reshape(nw, num_groups // nw, _GROUP))
  table_lin = _make_relayout(vocab)(word_embeddings.T)
  table = table_lin.reshape(vocab, dim)
  out5 = _make_gather(num_groups, seq, batch)(idx, table)
  return (out5.transpose(2, 4, 0, 1, 3)
          .reshape(batch, seq, dim))
